# baseline (speedup 1.0000x reference)
import functools

import numpy as np

import jax
import jax.numpy as jnp
from jax.experimental import pallas as pl
from jax.experimental.pallas import tpu as pltpu


def _block_kernel(x_ref, skip_ref, wt_ref, bt_ref, g1_ref, be1_ref,
                  perm_ref, w1u_ref, w1s_ref, b1_ref, w2_ref, b2_ref,
                  g2_ref, be2_ref,
                  out_ref,
                  z_ref, y_ref, acc1_ref, acc2_ref, sc1_ref, sh1_ref,
                  sc2_ref, sh2_ref,
                  *, Co, H, W, na, nc, GA, GC, cnt, eps):
    HW = H * W
    g = pl.program_id(0)
    offs = [(dy, dx) for dy in (-1, 0, 1) for dx in (-1, 0, 1)]

    @pl.when(g == 0)
    def _():
        acc1_ref[...] = jnp.zeros_like(acc1_ref)
        acc2_ref[...] = jnp.zeros_like(acc2_ref)

    @pl.when(g < GA)
    def _():
        w = wt_ref[...]
        b = bt_ref[...]
        acc = None
        for i in range(na):
            x = x_ref[i].astype(jnp.bfloat16)
            z = jnp.dot(w, x, preferred_element_type=jnp.float32)
            z = jnp.maximum(z + b, 0.0)
            st = jnp.concatenate(
                [jnp.sum(z, axis=1, keepdims=True),
                 jnp.sum(z * z, axis=1, keepdims=True)], axis=1)
            acc = st if acc is None else acc + st
            z_ref[g * na + i] = z
        acc1_ref[...] += acc

        @pl.when(g == GA - 1)
        def _():
            st = acc1_ref[...]
            stc = (st[0 * Co:1 * Co] + st[1 * Co:2 * Co]
                   + st[2 * Co:3 * Co] + st[3 * Co:4 * Co])
            mean = stc[:, 0:1] / cnt
            var = jnp.maximum(stc[:, 1:2] / cnt - mean * mean, 0.0)
            scale = g1_ref[...] * jax.lax.rsqrt(var + eps)
            shift = be1_ref[...] - mean * scale
            sc1_ref[...] = jnp.concatenate([scale] * 4, axis=0)
            sh1_ref[...] = jnp.concatenate([shift] * 4, axis=0)

    @pl.when((g >= GA) & (g < 2 * GA))
    def _():
        gb = g - GA

        col = jax.lax.broadcasted_iota(jnp.int32, (1, HW), 1)
        xi = col % W
        yi = col // W
        masks = {}
        for dy, dx in offs:
            if (dy, dx) == (0, 0):
                continue
            m = ((xi + dx >= 0) & (xi + dx < W) &
                 (yi + dy >= 0) & (yi + dy < H))
            masks[(dy, dx)] = m.astype(jnp.bfloat16)

        def patches(img):
            taps = []
            for dy, dx in offs:
                if (dy, dx) == (0, 0):
                    taps.append(img)
                    continue
                s = dy * W + dx
                shifted = pltpu.roll(img, shift=(-s) % HW, axis=1)
                taps.append(shifted * masks[(dy, dx)])
            return jnp.concatenate(taps, axis=0)

        sc1 = sc1_ref[...]
        sh1 = sh1_ref[...]
        acc = None
        for i in range(na):
            zf = z_ref[gb * na + i]
            zn = (zf * sc1 + sh1).astype(jnp.bfloat16)
            u = jnp.dot(zn[0 * Co:1 * Co], perm_ref[0],
                        preferred_element_type=jnp.float32)
            for p in range(1, 4):
                u = u + jnp.dot(zn[p * Co:(p + 1) * Co], perm_ref[p],
                                preferred_element_type=jnp.float32)
            u = u.astype(jnp.bfloat16)

            mid = (jnp.dot(w1u_ref[...], patches(u),
                           preferred_element_type=jnp.float32)
                   + jnp.dot(w1s_ref[...],
                             patches(skip_ref[i].astype(jnp.bfloat16)),
                             preferred_element_type=jnp.float32))
            mid = jnp.maximum(mid + b1_ref[...], 0.0).astype(jnp.bfloat16)
            y = jnp.dot(w2_ref[...], patches(mid),
                        preferred_element_type=jnp.float32)
            y = jnp.maximum(y + b2_ref[...], 0.0)

            y_ref[gb * na + i] = y
            st = jnp.concatenate(
                [jnp.sum(y, axis=1, keepdims=True),
                 jnp.sum(y * y, axis=1, keepdims=True)], axis=1)
            acc = st if acc is None else acc + st
        acc2_ref[...] += acc

        @pl.when(gb == GA - 1)
        def _():
            st = acc2_ref[...]
            mean = st[:, 0:1] / cnt
            var = jnp.maximum(st[:, 1:2] / cnt - mean * mean, 0.0)
            scale = g2_ref[...] * jax.lax.rsqrt(var + eps)
            sc2_ref[...] = scale
            sh2_ref[...] = be2_ref[...] - mean * scale

    @pl.when(g >= 2 * GA)
    def _():
        gc = g - 2 * GA
        sc = sc2_ref[...]
        sh = sh2_ref[...]
        for j in range(nc):
            out_ref[j] = y_ref[gc * nc + j] * sc + sh


def _upsample_block(x3, skip3, w_taps, b_taps, g1c, be1c, perm,
                    w1mu, w1ms, b1c, w2m, b2c, g2c, be2c, H, W, eps):
    N, Ci, HWc = x3.shape
    Cs = skip3.shape[1]
    Ko = w_taps.shape[0]
    Co = Ko // 4
    Cm = w1mu.shape[0]
    HW = H * W
    na = 2 if N % 2 == 0 else 1
    nc = 8 if N % 8 == 0 else 1
    GA = N // na
    GC = N // nc
    G = 2 * GA + GC
    cnt = N * HW

    body = functools.partial(_block_kernel, Co=Co, H=H, W=W, na=na, nc=nc,
                             GA=GA, GC=GC, cnt=cnt, eps=eps)
    return pl.pallas_call(
        body,
        grid=(G,),
        in_specs=[
            pl.BlockSpec((na, Ci, HWc),
                         lambda g: (jnp.minimum(g, GA - 1), 0, 0)),
            pl.BlockSpec((na, Cs, HW),
                         lambda g: (jnp.clip(g - GA, 0, GA - 1), 0, 0)),
            pl.BlockSpec((Ko, Ci), lambda g: (0, 0)),
            pl.BlockSpec((Ko, 1), lambda g: (0, 0)),
            pl.BlockSpec((Co, 1), lambda g: (0, 0)),
            pl.BlockSpec((Co, 1), lambda g: (0, 0)),
            pl.BlockSpec((4, HWc, HW), lambda g: (0, 0, 0)),
            pl.BlockSpec((Cm, 9 * Co), lambda g: (0, 0)),
            pl.BlockSpec((Cm, 9 * Cs), lambda g: (0, 0)),
            pl.BlockSpec((Cm, 1), lambda g: (0, 0)),
            pl.BlockSpec((Co, 9 * Cm), lambda g: (0, 0)),
            pl.BlockSpec((Co, 1), lambda g: (0, 0)),
            pl.BlockSpec((Co, 1), lambda g: (0, 0)),
            pl.BlockSpec((Co, 1), lambda g: (0, 0)),
        ],
        out_specs=pl.BlockSpec((nc, Co, HW),
                               lambda g: (jnp.clip(g - 2 * (N // na), 0,
                                                   N // nc - 1), 0, 0)),
        out_shape=jax.ShapeDtypeStruct((N, Co, HW), jnp.float32),
        scratch_shapes=[
            pltpu.VMEM((N, Ko, HWc), jnp.float32),
            pltpu.VMEM((N, Co, HW), jnp.float32),
            pltpu.VMEM((Ko, 2), jnp.float32),
            pltpu.VMEM((Co, 2), jnp.float32),
            pltpu.VMEM((Ko, 1), jnp.float32),
            pltpu.VMEM((Ko, 1), jnp.float32),
            pltpu.VMEM((Co, 1), jnp.float32),
            pltpu.VMEM((Co, 1), jnp.float32),
        ],
        compiler_params=pltpu.CompilerParams(
            dimension_semantics=("arbitrary",),
            vmem_limit_bytes=60 * 2 ** 20),
    )(x3, skip3, w_taps, b_taps, g1c, be1c, perm,
      w1mu, w1ms, b1c, w2m, b2c, g2c, be2c)


@functools.lru_cache(maxsize=4)
def _perm_mats(Hc, Wc):
    H2, W2 = 2 * Hc, 2 * Wc
    HWc = Hc * Wc
    P = np.zeros((4, HWc, H2 * W2), np.float32)
    hw = np.arange(HWc)
    h, w = hw // Wc, hw % Wc
    for dy in (0, 1):
        for dx in (0, 1):
            P[dy * 2 + dx, hw, (2 * h + dy) * W2 + 2 * w + dx] = 1.0
    return P


def kernel(x, skip, wt, bt, w1, b1, w2, b2, g1, be1, g2, be2):
    eps = 1e-5
    mxu_dtype = jnp.bfloat16
    x = x.astype(jnp.float32)
    skip = skip.astype(jnp.float32)
    N, Ci, Hc, Wc = x.shape
    _, Cs, H2, W2 = skip.shape
    Co = wt.shape[1]
    Cm = w1.shape[0]
    HWc, HW2 = Hc * Wc, H2 * W2

    x3 = x.reshape(N, Ci, HWc)
    skip3 = skip.reshape(N, Cs, HW2)

    w_taps = (jnp.transpose(wt, (2, 3, 1, 0))
              .reshape(4 * Co, Ci).astype(mxu_dtype))
    b_taps = jnp.tile(bt, 4).reshape(4 * Co, 1)
    perm = jnp.asarray(_perm_mats(Hc, Wc), dtype=mxu_dtype)
    w1mu = (jnp.transpose(w1[:, :Co], (0, 2, 3, 1))
            .reshape(Cm, 9 * Co).astype(mxu_dtype))
    w1ms = (jnp.transpose(w1[:, Co:], (0, 2, 3, 1))
            .reshape(Cm, 9 * Cs).astype(mxu_dtype))
    w2m = jnp.transpose(w2, (0, 2, 3, 1)).reshape(Co, 9 * Cm).astype(mxu_dtype)

    out = _upsample_block(x3, skip3, w_taps, b_taps,
                          g1.reshape(Co, 1), be1.reshape(Co, 1), perm,
                          w1mu, w1ms, b1.reshape(Cm, 1),
                          w2m, b2.reshape(Co, 1),
                          g2.reshape(Co, 1), be2.reshape(Co, 1),
                          H2, W2, eps)
    return out.reshape(N, Co, H2, W2)
